# baseline (speedup 1.0000x reference)
import jax
import jax.numpy as jnp
from jax.experimental import pallas as pl
from jax.experimental.pallas import tpu as pltpu

_PH, _PW = 7, 7
_SCALE = 0.0625
_ROWS = 40
_RB = 8
_NEG = float(jnp.finfo(jnp.float32).min)


def _roi_kernel(meta_ref, f_ref, o_ref, ptab_ref, t_ref):
    b = pl.program_id(0)
    rblk = pl.program_id(1)
    W = f_ref.shape[3]

    @pl.when(rblk == 0)
    def _build_tables():
        ptab_ref[0:W] = jnp.transpose(f_ref[0], (2, 1, 0))
        ptab_ref[W:2 * W - 1] = jnp.maximum(ptab_ref[0:W - 1], ptab_ref[1:W])
        ptab_ref[2 * W - 1] = ptab_ref[W - 1]
        ptab_ref[2 * W:3 * W - 2] = jnp.maximum(ptab_ref[W:2 * W - 2],
                                                ptab_ref[W + 2:2 * W])
        ptab_ref[3 * W - 2:3 * W] = ptab_ref[2 * W - 2:2 * W]
        ptab_ref[3 * W:4 * W - 4] = jnp.maximum(ptab_ref[2 * W:3 * W - 4],
                                                ptab_ref[2 * W + 4:3 * W])
        ptab_ref[4 * W - 4:4 * W] = ptab_ref[3 * W - 4:3 * W]

    iot8 = jax.lax.broadcasted_iota(jnp.int32, (8, 1), 0)

    for u in range(_RB):
        rr = rblk * _RB + u
        y0a = pl.multiple_of(meta_ref[b, rr, 0], 8)
        rows_sl = pl.ds(y0a, _ROWS)

        for j in range(_PW):
            acc = jnp.maximum(ptab_ref[meta_ref[b, rr, 1 + j], rows_sl, :],
                              ptab_ref[meta_ref[b, rr, 8 + j], rows_sl, :])
            acc = jnp.where(meta_ref[b, rr, 15 + j] > 0, acc, _NEG)
            t_ref[pl.ds(35 * u + 5 * j, 5)] = acc.reshape(5, 8, 128)

        vals = []
        for i in range(_PH):
            ga = meta_ref[b, rr, 22 + i]
            gb = meta_ref[b, rr, 29 + i]
            lo = meta_ref[b, rr, 36 + i]
            hi = meta_ref[b, rr, 43 + i]
            ma = (iot8 + 8 * ga >= lo) & (iot8 + 8 * ga < hi)
            mb = (iot8 + 8 * gb >= lo) & (iot8 + 8 * gb < hi)
            for j in range(_PW):
                va = jnp.where(ma, t_ref[35 * u + 5 * j + ga], _NEG)
                vb = jnp.where(mb, t_ref[35 * u + 5 * j + gb], _NEG)
                vals.append(jnp.maximum(va, vb).max(axis=0))
        for i in range(_PH):
            o_ref[0, u, _PW * i:_PW * (i + 1), :] = jnp.stack(
                vals[_PW * i:_PW * (i + 1)], axis=0)


def _make_meta(rois, H, W):
    r = rois.astype(jnp.float32)
    x = _SCALE * r[..., 1]
    y = _SCALE * r[..., 2]
    w = jnp.maximum(_SCALE * r[..., 3], 1.0)
    h = jnp.maximum(_SCALE * r[..., 4], 1.0)
    y = jnp.where(y >= H, H - h, y)
    x = jnp.where(x >= W, W - w, x)
    y0 = jnp.floor(y).astype(jnp.int32)
    x0 = jnp.floor(x).astype(jnp.int32)
    hc = jnp.maximum(jnp.minimum(jnp.floor(y + h).astype(jnp.int32), H) - y0, 1)
    wc = jnp.maximum(jnp.minimum(jnp.floor(x + w).astype(jnp.int32), W) - x0, 1)

    y0a = jnp.clip((y0 // 8) * 8, 0, H - _ROWS)
    dy = y0 - y0a

    jj = jnp.arange(_PW)
    ws = (jj * wc[..., None]) // _PW
    we = -((-(jj + 1) * wc[..., None]) // _PW)
    L = we - ws
    k = ((L >= 2).astype(jnp.int32) + (L >= 4).astype(jnp.int32)
         + (L >= 8).astype(jnp.int32))
    pw2 = jnp.left_shift(1, k)
    ia = k * W + jnp.clip(x0[..., None] + ws, 0, W - 1)
    ib = k * W + jnp.clip(x0[..., None] + we - pw2, 0, W - 1)
    wflag = (we > ws).astype(jnp.int32)

    ii = jnp.arange(_PH)
    bs = (ii * hc[..., None]) // _PH
    be = -((-(ii + 1) * hc[..., None]) // _PH)
    lo = bs + dy[..., None]
    hi = be + dy[..., None]
    ga = lo // 8
    gb = jnp.maximum((hi - 1) // 8, ga)

    B, R = rois.shape[:2]
    pad = jnp.zeros((B, R, 6), jnp.int32)
    return jnp.concatenate(
        [y0a[..., None], ia, ib, wflag, ga, gb, lo, hi, pad],
        axis=-1)


def kernel(features, rois):
    B, C, H, W = features.shape
    R = rois.shape[1]
    meta = _make_meta(rois, H, W)
    out = pl.pallas_call(
        _roi_kernel,
        out_shape=jax.ShapeDtypeStruct((B, R, _PH * _PW, C), jnp.float32),
        grid_spec=pltpu.PrefetchScalarGridSpec(
            num_scalar_prefetch=1,
            grid=(B, R // _RB),
            in_specs=[pl.BlockSpec((1, C, H, W), lambda b, r, meta_s: (b, 0, 0, 0))],
            out_specs=pl.BlockSpec((1, _RB, _PH * _PW, C),
                                   lambda b, r, meta_s: (b, r, 0, 0)),
            scratch_shapes=[pltpu.VMEM((4 * W, H, C), jnp.float32),
                            pltpu.VMEM((35 * _RB, 8, C), jnp.float32)],
        ),
        compiler_params=pltpu.CompilerParams(
            dimension_semantics=("parallel", "arbitrary"),
        ),
        name="roi_maxpool",
    )(meta, features)
    return out.transpose(0, 1, 3, 2).reshape(B, R, C, _PH, _PW)


# speedup vs baseline: 1.2013x; 1.2013x over previous
import jax
import jax.numpy as jnp
from jax.experimental import pallas as pl
from jax.experimental.pallas import tpu as pltpu

_PH, _PW = 7, 7
_SCALE = 0.0625
_ROWS = 40
_RB = 16
_NEG = float(jnp.finfo(jnp.float32).min)


def _roi_kernel(meta_ref, f_ref, o_ref, ptab_ref, t_ref):
    b = pl.program_id(0)
    rblk = pl.program_id(1)
    W = f_ref.shape[1]

    @pl.when(rblk == 0)
    def _build_tables():
        ptab_ref[0:W] = f_ref[0]
        ptab_ref[W:2 * W - 1] = jnp.maximum(f_ref[0, 0:W - 1], f_ref[0, 1:W])
        ptab_ref[2 * W - 1] = f_ref[0, W - 1]
        ptab_ref[2 * W:3 * W - 2] = jnp.maximum(ptab_ref[W:2 * W - 2],
                                                ptab_ref[W + 2:2 * W])
        ptab_ref[3 * W - 2:3 * W] = ptab_ref[2 * W - 2:2 * W]
        ptab_ref[3 * W:4 * W - 4] = jnp.maximum(ptab_ref[2 * W:3 * W - 4],
                                                ptab_ref[2 * W + 4:3 * W])
        ptab_ref[4 * W - 4:4 * W] = ptab_ref[3 * W - 4:3 * W]

    iot8 = jax.lax.broadcasted_iota(jnp.int32, (8, 1), 0)

    for u in range(_RB):
        rr = rblk * _RB + u
        y0a = pl.multiple_of(meta_ref[b, rr, 0], 8)
        rows_sl = pl.ds(y0a, _ROWS)

        for j in range(_PW):
            acc = jnp.maximum(ptab_ref[meta_ref[b, rr, 1 + j], rows_sl, :],
                              ptab_ref[meta_ref[b, rr, 8 + j], rows_sl, :])
            acc = jnp.where(meta_ref[b, rr, 15 + j] > 0, acc, _NEG)
            t_ref[pl.ds(35 * u + 5 * j, 5)] = acc.reshape(5, 8, 128)

        vals = []
        for i in range(_PH):
            ga = meta_ref[b, rr, 22 + i]
            gb = meta_ref[b, rr, 29 + i]
            lo = meta_ref[b, rr, 36 + i]
            hi = meta_ref[b, rr, 43 + i]
            ma = (iot8 + 8 * ga >= lo) & (iot8 + 8 * ga < hi)
            mb = (iot8 + 8 * gb >= lo) & (iot8 + 8 * gb < hi)
            for j in range(_PW):
                va = jnp.where(ma, t_ref[35 * u + 5 * j + ga], _NEG)
                vb = jnp.where(mb, t_ref[35 * u + 5 * j + gb], _NEG)
                vals.append(jnp.maximum(va, vb).max(axis=0))
        for i in range(_PH):
            o_ref[0, u, _PW * i:_PW * (i + 1), :] = jnp.stack(
                vals[_PW * i:_PW * (i + 1)], axis=0)


def _make_meta(rois, H, W):
    r = rois.astype(jnp.float32)
    x = _SCALE * r[..., 1]
    y = _SCALE * r[..., 2]
    w = jnp.maximum(_SCALE * r[..., 3], 1.0)
    h = jnp.maximum(_SCALE * r[..., 4], 1.0)
    y = jnp.where(y >= H, H - h, y)
    x = jnp.where(x >= W, W - w, x)
    y0 = jnp.floor(y).astype(jnp.int32)
    x0 = jnp.floor(x).astype(jnp.int32)
    hc = jnp.maximum(jnp.minimum(jnp.floor(y + h).astype(jnp.int32), H) - y0, 1)
    wc = jnp.maximum(jnp.minimum(jnp.floor(x + w).astype(jnp.int32), W) - x0, 1)

    y0a = jnp.clip((y0 // 8) * 8, 0, H - _ROWS)
    dy = y0 - y0a

    jj = jnp.arange(_PW)
    ws = (jj * wc[..., None]) // _PW
    we = -((-(jj + 1) * wc[..., None]) // _PW)
    L = we - ws
    k = ((L >= 2).astype(jnp.int32) + (L >= 4).astype(jnp.int32)
         + (L >= 8).astype(jnp.int32))
    pw2 = jnp.left_shift(1, k)
    ia = k * W + jnp.clip(x0[..., None] + ws, 0, W - 1)
    ib = k * W + jnp.clip(x0[..., None] + we - pw2, 0, W - 1)
    wflag = (we > ws).astype(jnp.int32)

    ii = jnp.arange(_PH)
    bs = (ii * hc[..., None]) // _PH
    be = -((-(ii + 1) * hc[..., None]) // _PH)
    lo = bs + dy[..., None]
    hi = be + dy[..., None]
    ga = lo // 8
    gb = jnp.maximum((hi - 1) // 8, ga)

    B, R = rois.shape[:2]
    pad = jnp.zeros((B, R, 6), jnp.int32)
    return jnp.concatenate(
        [y0a[..., None], ia, ib, wflag, ga, gb, lo, hi, pad],
        axis=-1)


def kernel(features, rois):
    B, C, H, W = features.shape
    R = rois.shape[1]
    f_t = jnp.transpose(features, (0, 3, 2, 1))
    meta = _make_meta(rois, H, W)
    out = pl.pallas_call(
        _roi_kernel,
        out_shape=jax.ShapeDtypeStruct((B, R, _PH * _PW, C), jnp.float32),
        grid_spec=pltpu.PrefetchScalarGridSpec(
            num_scalar_prefetch=1,
            grid=(B, R // _RB),
            in_specs=[pl.BlockSpec((1, W, H, C), lambda b, r, meta_s: (b, 0, 0, 0))],
            out_specs=pl.BlockSpec((1, _RB, _PH * _PW, C),
                                   lambda b, r, meta_s: (b, r, 0, 0)),
            scratch_shapes=[pltpu.VMEM((4 * W, H, C), jnp.float32),
                            pltpu.VMEM((35 * _RB, 8, C), jnp.float32)],
        ),
        compiler_params=pltpu.CompilerParams(
            dimension_semantics=("parallel", "arbitrary"),
        ),
        name="roi_maxpool",
    )(meta, f_t)
    return out.transpose(0, 1, 3, 2).reshape(B, R, C, _PH, _PW)


# speedup vs baseline: 1.2205x; 1.0160x over previous
import jax
import jax.numpy as jnp
from jax.experimental import pallas as pl
from jax.experimental.pallas import tpu as pltpu

_PH, _PW = 7, 7
_SCALE = 0.0625
_ROWS = 40
_RB = 32
_NEG = float(jnp.finfo(jnp.float32).min)


def _roi_kernel(meta_ref, f_ref, o_ref, ptab_ref, t_ref):
    b = pl.program_id(0)
    rblk = pl.program_id(1)
    W = f_ref.shape[1]

    @pl.when(rblk == 0)
    def _build_tables():
        ptab_ref[0:W] = f_ref[0]
        ptab_ref[W:2 * W - 1] = jnp.maximum(f_ref[0, 0:W - 1], f_ref[0, 1:W])
        ptab_ref[2 * W - 1] = f_ref[0, W - 1]
        ptab_ref[2 * W:3 * W - 2] = jnp.maximum(ptab_ref[W:2 * W - 2],
                                                ptab_ref[W + 2:2 * W])
        ptab_ref[3 * W - 2:3 * W] = ptab_ref[2 * W - 2:2 * W]
        ptab_ref[3 * W:4 * W - 4] = jnp.maximum(ptab_ref[2 * W:3 * W - 4],
                                                ptab_ref[2 * W + 4:3 * W])
        ptab_ref[4 * W - 4:4 * W] = ptab_ref[3 * W - 4:3 * W]

    iot8 = jax.lax.broadcasted_iota(jnp.int32, (8, 1), 0)

    for u in range(_RB):
        rr = rblk * _RB + u
        y0a = pl.multiple_of(meta_ref[b, rr, 0], 8)
        rows_sl = pl.ds(y0a, _ROWS)

        for j in range(_PW):
            acc = jnp.maximum(ptab_ref[meta_ref[b, rr, 1 + j], rows_sl, :],
                              ptab_ref[meta_ref[b, rr, 8 + j], rows_sl, :])
            acc = jnp.where(meta_ref[b, rr, 15 + j] > 0, acc, _NEG)
            t_ref[pl.ds(35 * u + 5 * j, 5)] = acc.reshape(5, 8, 128)

        vals = []
        for i in range(_PH):
            ga = meta_ref[b, rr, 22 + i]
            gb = meta_ref[b, rr, 29 + i]
            lo = meta_ref[b, rr, 36 + i]
            hi = meta_ref[b, rr, 43 + i]
            ma = (iot8 + 8 * ga >= lo) & (iot8 + 8 * ga < hi)
            mb = (iot8 + 8 * gb >= lo) & (iot8 + 8 * gb < hi)
            for j in range(_PW):
                va = jnp.where(ma, t_ref[35 * u + 5 * j + ga], _NEG)
                vb = jnp.where(mb, t_ref[35 * u + 5 * j + gb], _NEG)
                vals.append(jnp.maximum(va, vb).max(axis=0))
        for i in range(_PH):
            o_ref[0, u, _PW * i:_PW * (i + 1), :] = jnp.stack(
                vals[_PW * i:_PW * (i + 1)], axis=0)


def _make_meta(rois, H, W):
    r = rois.astype(jnp.float32)
    x = _SCALE * r[..., 1]
    y = _SCALE * r[..., 2]
    w = jnp.maximum(_SCALE * r[..., 3], 1.0)
    h = jnp.maximum(_SCALE * r[..., 4], 1.0)
    y = jnp.where(y >= H, H - h, y)
    x = jnp.where(x >= W, W - w, x)
    y0 = jnp.floor(y).astype(jnp.int32)
    x0 = jnp.floor(x).astype(jnp.int32)
    hc = jnp.maximum(jnp.minimum(jnp.floor(y + h).astype(jnp.int32), H) - y0, 1)
    wc = jnp.maximum(jnp.minimum(jnp.floor(x + w).astype(jnp.int32), W) - x0, 1)

    y0a = jnp.clip((y0 // 8) * 8, 0, H - _ROWS)
    dy = y0 - y0a

    jj = jnp.arange(_PW)
    ws = (jj * wc[..., None]) // _PW
    we = -((-(jj + 1) * wc[..., None]) // _PW)
    L = we - ws
    k = ((L >= 2).astype(jnp.int32) + (L >= 4).astype(jnp.int32)
         + (L >= 8).astype(jnp.int32))
    pw2 = jnp.left_shift(1, k)
    ia = k * W + jnp.clip(x0[..., None] + ws, 0, W - 1)
    ib = k * W + jnp.clip(x0[..., None] + we - pw2, 0, W - 1)
    wflag = (we > ws).astype(jnp.int32)

    ii = jnp.arange(_PH)
    bs = (ii * hc[..., None]) // _PH
    be = -((-(ii + 1) * hc[..., None]) // _PH)
    lo = bs + dy[..., None]
    hi = be + dy[..., None]
    ga = lo // 8
    gb = jnp.maximum((hi - 1) // 8, ga)

    B, R = rois.shape[:2]
    pad = jnp.zeros((B, R, 6), jnp.int32)
    return jnp.concatenate(
        [y0a[..., None], ia, ib, wflag, ga, gb, lo, hi, pad],
        axis=-1)


def kernel(features, rois):
    B, C, H, W = features.shape
    R = rois.shape[1]
    f_t = jnp.transpose(features, (0, 3, 2, 1))
    meta = _make_meta(rois, H, W)
    out = pl.pallas_call(
        _roi_kernel,
        out_shape=jax.ShapeDtypeStruct((B, R, _PH * _PW, C), jnp.float32),
        grid_spec=pltpu.PrefetchScalarGridSpec(
            num_scalar_prefetch=1,
            grid=(B, R // _RB),
            in_specs=[pl.BlockSpec((1, W, H, C), lambda b, r, meta_s: (b, 0, 0, 0))],
            out_specs=pl.BlockSpec((1, _RB, _PH * _PW, C),
                                   lambda b, r, meta_s: (b, r, 0, 0)),
            scratch_shapes=[pltpu.VMEM((4 * W, H, C), jnp.float32),
                            pltpu.VMEM((35 * _RB, 8, C), jnp.float32)],
        ),
        compiler_params=pltpu.CompilerParams(
            dimension_semantics=("parallel", "arbitrary"),
        ),
        name="roi_maxpool",
    )(meta, f_t)
    return out.transpose(0, 1, 3, 2).reshape(B, R, C, _PH, _PW)
